# baseline (speedup 1.0000x reference)
import jax
import jax.numpy as jnp
from jax.experimental import pallas as pl

_TOP_K = 8
_T = 1024
_C = 128


def _gate_kernel(hs_ref, w_ref, idx_ref, wgt_ref):
    logits_t = jax.lax.dot_general(
        w_ref[...], hs_ref[...], (((1,), (1,)), ((), ())),
        preferred_element_type=jnp.float32,
    )
    e = logits_t.shape[0]
    iota_f = jax.lax.broadcasted_iota(jnp.int32, (e, _C), 0).astype(jnp.float32)
    kiota_f = jax.lax.broadcasted_iota(jnp.int32, (_TOP_K, _C), 0).astype(
        jnp.float32
    )
    for c in range(_T // _C):
        cur = logits_t[:, c * _C : (c + 1) * _C]
        vtop = jnp.zeros((_TOP_K, _C), jnp.float32)
        itop = jnp.zeros((_TOP_K, _C), jnp.float32)
        for k in range(_TOP_K):
            m = jnp.max(cur, axis=0, keepdims=True)
            is_max = cur == m
            i = jnp.min(
                jnp.where(is_max, iota_f, float(e)), axis=0, keepdims=True
            )
            vtop = jnp.where(kiota_f == float(k), m, vtop)
            itop = jnp.where(kiota_f == float(k), i, itop)
            cur = jnp.where(iota_f == i, -jnp.inf, cur)
        ex = jnp.exp(vtop - vtop[:1, :])
        wgt = ex / jnp.sum(ex, axis=0, keepdims=True)
        idx_ref[c * _C : (c + 1) * _C, :] = itop.astype(jnp.int32).T
        wgt_ref[c * _C : (c + 1) * _C, :] = wgt.T


def kernel(hidden_states, weight):
    b, s, h = hidden_states.shape
    hs = hidden_states.reshape(-1, h)
    n = hs.shape[0]
    e = weight.shape[0]
    grid = n // _T
    idx, wgt = pl.pallas_call(
        _gate_kernel,
        grid=(grid,),
        in_specs=[
            pl.BlockSpec((_T, h), lambda i: (i, 0)),
            pl.BlockSpec((e, h), lambda i: (0, 0)),
        ],
        out_specs=[
            pl.BlockSpec((_T, _TOP_K), lambda i: (i, 0)),
            pl.BlockSpec((_T, _TOP_K), lambda i: (i, 0)),
        ],
        out_shape=[
            jax.ShapeDtypeStruct((n, _TOP_K), jnp.int32),
            jax.ShapeDtypeStruct((n, _TOP_K), jnp.float32),
        ],
    )(hs, weight)
    return idx, wgt


# speedup vs baseline: 1.0231x; 1.0231x over previous
import jax
import jax.numpy as jnp
from jax.experimental import pallas as pl

_TOP_K = 8
_T = 2048
_C = 256


def _gate_kernel(hs_ref, w_ref, idx_ref, wgt_ref):
    logits_t = jax.lax.dot_general(
        w_ref[...], hs_ref[...], (((1,), (1,)), ((), ())),
        preferred_element_type=jnp.float32,
    )
    e = logits_t.shape[0]
    iota_f = jax.lax.broadcasted_iota(jnp.int32, (e, _C), 0).astype(jnp.float32)
    kiota_f = jax.lax.broadcasted_iota(jnp.int32, (_TOP_K, _C), 0).astype(
        jnp.float32
    )
    for c in range(_T // _C):
        cur = logits_t[:, c * _C : (c + 1) * _C]
        vtop = jnp.zeros((_TOP_K, _C), jnp.float32)
        itop = jnp.zeros((_TOP_K, _C), jnp.float32)
        for k in range(_TOP_K):
            m = jnp.max(cur, axis=0, keepdims=True)
            is_max = cur == m
            i = jnp.min(
                jnp.where(is_max, iota_f, float(e)), axis=0, keepdims=True
            )
            vtop = jnp.where(kiota_f == float(k), m, vtop)
            itop = jnp.where(kiota_f == float(k), i, itop)
            cur = jnp.where(iota_f == i, -jnp.inf, cur)
        ex = jnp.exp(vtop - vtop[:1, :])
        wgt = ex / jnp.sum(ex, axis=0, keepdims=True)
        idx_ref[c * _C : (c + 1) * _C, :] = itop.astype(jnp.int32).T
        wgt_ref[c * _C : (c + 1) * _C, :] = wgt.T


def kernel(hidden_states, weight):
    b, s, h = hidden_states.shape
    hs = hidden_states.reshape(-1, h)
    n = hs.shape[0]
    e = weight.shape[0]
    grid = n // _T
    idx, wgt = pl.pallas_call(
        _gate_kernel,
        grid=(grid,),
        in_specs=[
            pl.BlockSpec((_T, h), lambda i: (i, 0)),
            pl.BlockSpec((e, h), lambda i: (0, 0)),
        ],
        out_specs=[
            pl.BlockSpec((_T, _TOP_K), lambda i: (i, 0)),
            pl.BlockSpec((_T, _TOP_K), lambda i: (i, 0)),
        ],
        out_shape=[
            jax.ShapeDtypeStruct((n, _TOP_K), jnp.int32),
            jax.ShapeDtypeStruct((n, _TOP_K), jnp.float32),
        ],
    )(hs, weight)
    return idx, wgt
